# baseline (speedup 1.0000x reference)
import functools
import math

import jax
import jax.numpy as jnp
from jax import lax
from jax.experimental import pallas as pl
from jax.experimental.pallas import tpu as pltpu
from jax.experimental.pallas import tpu_sc as plsc

_N = 16384
_SIZE = 1000
_SMOOTH = 0.1
_CONF = 1.0 - _SMOOTH
_S = _SMOOTH / (_SIZE - 2)
_K = (_SIZE - 2) * _S * math.log(_S) + _CONF * math.log(_CONF)

_INFO = plsc.get_sparse_core_info()
_NC, _NS, _L = _INFO.num_cores, _INFO.num_subcores, _INFO.num_lanes
_NW = _NC * _NS
_RPW = _N // _NW
_TR = 16
_NT = _RPW // _TR
_UNROLL = 8


@functools.partial(
    pl.kernel,
    mesh=plsc.VectorSubcoreMesh(core_axis_name="c", subcore_axis_name="s"),
    out_type=jax.ShapeDtypeStruct((_NW * _L,), jnp.float32),
    scratch_types=[
        pltpu.VMEM((_RPW,), jnp.int32),
        pltpu.VMEM((_TR, _SIZE), jnp.float32),
        pltpu.VMEM((_L,), jnp.float32),
        pltpu.SemaphoreType.DMA,
    ],
    compiler_params=pltpu.CompilerParams(
        use_tc_tiling_on_sc=False, needs_layout_passes=False),
)
def _sc_loss(x_hbm, tgt_hbm, out_hbm, tgt_v, buf, res_v, sem):
    wid = lax.axis_index("s") * _NC + lax.axis_index("c")
    base = wid * _RPW
    pltpu.sync_copy(tgt_hbm.at[pl.ds(base, _RPW)], tgt_v)
    iota16 = lax.iota(jnp.int32, _L)
    zeros16 = jnp.zeros((_L,), jnp.int32)
    zf = jnp.zeros((_L,), jnp.float32)

    def tile_body(t, accs):
        acc_rs, acc_pick, acc_x0, acc_cnt = accs
        row0 = base + t * _TR
        pltpu.sync_copy(x_hbm.at[pl.ds(row0, _TR)], buf)

        def col_body(cb, rs):
            for u in range(_UNROLL):
                c = cb * _UNROLL + u
                cs = jnp.full((_L,), c, jnp.int32)
                rs = rs + plsc.load_gather(buf, [iota16, cs])
            return rs

        rowsum = lax.fori_loop(0, _SIZE // _UNROLL, col_body, zf)
        tgt16 = tgt_v[pl.ds(t * _TR, _L)]
        valid = tgt16 != 0
        pick = plsc.load_gather(buf, [iota16, tgt16])
        x0v = plsc.load_gather(buf, [iota16, zeros16])
        acc_rs = acc_rs + jnp.where(valid, rowsum, zf)
        acc_pick = acc_pick + jnp.where(valid, pick, zf)
        acc_x0 = acc_x0 + jnp.where(valid, x0v, zf)
        acc_cnt = acc_cnt + jnp.where(valid, jnp.full((_L,), 1.0, jnp.float32), zf)
        return (acc_rs, acc_pick, acc_x0, acc_cnt)

    acc_rs, acc_pick, acc_x0, acc_cnt = lax.fori_loop(
        0, _NT, tile_body, (zf, zf, zf, zf))
    res_v[...] = (_K * acc_cnt - _S * acc_rs + _S * acc_x0
                  - (_CONF - _S) * acc_pick)
    pltpu.sync_copy(res_v, out_hbm.at[pl.ds(wid * _L, _L)])


def kernel(x, target):
    parts = _sc_loss(x, target.astype(jnp.int32))
    return jnp.sum(parts)
